# Initial kernel scaffold; baseline (speedup 1.0000x reference)
import jax
import jax.numpy as jnp
from jax.experimental import pallas as pl


def kernel(origins, directions, appearance_embedding, centroids, W1, b1, W2, b2, W3, b3):
    raise NotImplementedError("write your pallas kernel here")


# speedup vs baseline: 13.3452x; 13.3452x over previous
import jax
import jax.numpy as jnp
from jax.experimental import pallas as pl

E = 16
APP_DIM = 32
HID = 64
IN_PAD = 64
OUT_PAD = 8
BLK = 512


def _moe_body(o_ref, x_ref, cmat_ref, w1_ref, b1_ref, w2_ref, b2_ref,
              w3_ref, b3_ref, out_ref):
    o = o_ref[...]
    x = x_ref[...]

    score = jnp.zeros((BLK, 128), jnp.float32)
    for k in range(3):
        d = o[:, k:k + 1] - cmat_ref[k:k + 1, :]
        score = score + d * d
    cluster = jnp.argmin(score, axis=1).astype(jnp.int32)

    acc = jnp.zeros((BLK, OUT_PAD), jnp.float32)
    for e in range(E):
        h = jnp.maximum(
            jnp.dot(x, w1_ref[e], preferred_element_type=jnp.float32)
            + b1_ref[e:e + 1, :], 0.0)
        h = jnp.maximum(
            jnp.dot(h, w2_ref[e], preferred_element_type=jnp.float32)
            + b2_ref[e:e + 1, :], 0.0)
        r = jnp.dot(h, w3_ref[e], preferred_element_type=jnp.float32) \
            + b3_ref[e:e + 1, :]
        m = (cluster == e).astype(jnp.float32)[:, None]
        acc = acc + m * r
    out_ref[...] = jax.nn.sigmoid(acc)


def kernel(origins, directions, appearance_embedding, centroids,
           W1, b1, W2, b2, W3, b3):
    n = origins.shape[0]
    o = origins[:, 0, :]
    d = directions[:, 0, :]
    a = appearance_embedding[:, 0, :]
    in_dim = 3 + a.shape[1]

    o_pad = jnp.pad(o, ((0, 0), (0, 8 - 3)))
    x = jnp.concatenate([d, a], axis=-1)
    x_pad = jnp.pad(x, ((0, 0), (0, IN_PAD - in_dim)))

    cmat = jnp.full((8, 128), 1e9, jnp.float32)
    cmat = cmat.at[:3, :E].set(centroids.T)

    w1p = jnp.pad(W1, ((0, 0), (0, IN_PAD - in_dim), (0, 0)))
    w3p = jnp.pad(W3, ((0, 0), (0, 0), (0, OUT_PAD - 3)))
    b3p = jnp.pad(b3, ((0, 0), (0, OUT_PAD - 3)))

    grid = (n // BLK,)
    out = pl.pallas_call(
        _moe_body,
        grid=grid,
        in_specs=[
            pl.BlockSpec((BLK, 8), lambda i: (i, 0)),
            pl.BlockSpec((BLK, IN_PAD), lambda i: (i, 0)),
            pl.BlockSpec((8, 128), lambda i: (0, 0)),
            pl.BlockSpec((E, IN_PAD, HID), lambda i: (0, 0, 0)),
            pl.BlockSpec((E, HID), lambda i: (0, 0)),
            pl.BlockSpec((E, HID, HID), lambda i: (0, 0, 0)),
            pl.BlockSpec((E, HID), lambda i: (0, 0)),
            pl.BlockSpec((E, HID, OUT_PAD), lambda i: (0, 0, 0)),
            pl.BlockSpec((E, OUT_PAD), lambda i: (0, 0)),
        ],
        out_specs=pl.BlockSpec((BLK, OUT_PAD), lambda i: (i, 0)),
        out_shape=jax.ShapeDtypeStruct((n, OUT_PAD), jnp.float32),
    )(o_pad, x_pad, cmat, w1p, b1, W2, b2, w3p, b3p)
    return out[:, :3]


# speedup vs baseline: 16.4667x; 1.2339x over previous
import jax
import jax.numpy as jnp
from jax.experimental import pallas as pl

E = 16
HID = 64
IN_PAD = 64
OUT_PAD = 8
WIDE = E * HID
AUG = 128 + WIDE
BLK = 512


def _moe_body(o_ref, x_ref, cmat_ref, w1_ref, b1_ref, w2a_ref, w3a_ref,
              out_ref):
    o = o_ref[...]
    x = x_ref[...]

    score = jnp.zeros((BLK, 128), jnp.float32)
    for k in range(3):
        d = o[:, k:k + 1] - cmat_ref[k:k + 1, :]
        score = score + d * d
    cluster = jnp.argmin(score, axis=1).astype(jnp.int32)[:, None]

    lane128 = jax.lax.broadcasted_iota(jnp.int32, (BLK, 128), 1)
    onehot = (lane128 == cluster).astype(jnp.float32)
    eidx = jax.lax.broadcasted_iota(jnp.int32, (BLK, WIDE), 1) // HID
    mask = (eidx == cluster).astype(jnp.float32)

    h1 = jnp.maximum(
        jnp.dot(x, w1_ref[...], preferred_element_type=jnp.float32)
        + b1_ref[...], 0.0)
    g1 = jnp.concatenate([onehot, h1 * mask], axis=1)
    h2 = jnp.maximum(
        jnp.dot(g1, w2a_ref[...], preferred_element_type=jnp.float32), 0.0)
    h2t = jnp.concatenate([h2] * E, axis=1)
    g2 = jnp.concatenate([onehot, h2t * mask], axis=1)
    r = jnp.dot(g2, w3a_ref[...], preferred_element_type=jnp.float32)
    out_ref[...] = jax.nn.sigmoid(r)


def kernel(origins, directions, appearance_embedding, centroids,
           W1, b1, W2, b2, W3, b3):
    n = origins.shape[0]
    o = origins[:, 0, :]
    d = directions[:, 0, :]
    a = appearance_embedding[:, 0, :]
    in_dim = 3 + a.shape[1]

    o_pad = jnp.pad(o, ((0, 0), (0, 8 - 3)))
    x = jnp.concatenate([d, a], axis=-1)
    x_pad = jnp.pad(x, ((0, 0), (0, IN_PAD - in_dim)))

    cmat = jnp.full((8, 128), 1e9, jnp.float32)
    cmat = cmat.at[:3, :E].set(centroids.T)

    w1cat = jnp.pad(W1, ((0, 0), (0, IN_PAD - in_dim), (0, 0))) \
        .transpose(1, 0, 2).reshape(IN_PAD, WIDE)
    b1cat = b1.reshape(1, WIDE)

    w2aug = jnp.concatenate(
        [jnp.pad(b2, ((0, 128 - E), (0, 0))), W2.reshape(WIDE, HID)], axis=0)
    w3aug = jnp.concatenate(
        [jnp.pad(b3, ((0, 128 - E), (0, OUT_PAD - 3))),
         jnp.pad(W3, ((0, 0), (0, 0), (0, OUT_PAD - 3))).reshape(WIDE, OUT_PAD)],
        axis=0)

    grid = (n // BLK,)
    out = pl.pallas_call(
        _moe_body,
        grid=grid,
        in_specs=[
            pl.BlockSpec((BLK, 8), lambda i: (i, 0)),
            pl.BlockSpec((BLK, IN_PAD), lambda i: (i, 0)),
            pl.BlockSpec((8, 128), lambda i: (0, 0)),
            pl.BlockSpec((IN_PAD, WIDE), lambda i: (0, 0)),
            pl.BlockSpec((1, WIDE), lambda i: (0, 0)),
            pl.BlockSpec((AUG, HID), lambda i: (0, 0)),
            pl.BlockSpec((AUG, OUT_PAD), lambda i: (0, 0)),
        ],
        out_specs=pl.BlockSpec((BLK, OUT_PAD), lambda i: (i, 0)),
        out_shape=jax.ShapeDtypeStruct((n, OUT_PAD), jnp.float32),
    )(o_pad, x_pad, cmat, w1cat, b1cat, w2aug, w3aug)
    return out[:, :3]


# speedup vs baseline: 24.0402x; 1.4599x over previous
import jax
import jax.numpy as jnp
from jax.experimental import pallas as pl

E = 16
HID = 64
WIDE = E * HID
OUT_G = 8
BLK = 512


def _moe_body(o_ref, d_ref, a_ref, cmat_ref, w1d_ref, w1a_ref, b1_ref,
              w2s_ref, b2p_ref, w3c_ref, b3c_ref, out_ref):
    o = o_ref[...]
    d = d_ref[...]
    a = a_ref[...]

    score = jnp.zeros((BLK, 128), jnp.float32)
    for k in range(3):
        dd = o[:, k:k + 1] - cmat_ref[k:k + 1, :]
        score = score + dd * dd
    cluster = jnp.argmin(score, axis=1).astype(jnp.int32)[:, None]

    lane128 = jax.lax.broadcasted_iota(jnp.int32, (BLK, 128), 1)
    onehot = (lane128 == cluster).astype(jnp.float32)
    eidx = jax.lax.broadcasted_iota(jnp.int32, (BLK, WIDE), 1) // HID
    mask = (eidx == cluster).astype(jnp.float32)

    h1 = jnp.maximum(
        jnp.dot(d, w1d_ref[...], preferred_element_type=jnp.float32)
        + jnp.dot(a, w1a_ref[...], preferred_element_type=jnp.float32)
        + b1_ref[...], 0.0)
    h2 = jnp.maximum(
        jnp.dot(h1 * mask, w2s_ref[...], preferred_element_type=jnp.float32)
        + jnp.dot(onehot, b2p_ref[...], preferred_element_type=jnp.float32),
        0.0)
    r = jnp.dot(h2, w3c_ref[...], preferred_element_type=jnp.float32) \
        + b3c_ref[...]
    mask8 = (lane128 // OUT_G == cluster).astype(jnp.float32)
    rm = r * mask8
    rm = rm[:, :64] + rm[:, 64:]
    rm = rm[:, :32] + rm[:, 32:]
    rm = rm[:, :16] + rm[:, 16:]
    rm = rm[:, :8] + rm[:, 8:]
    out_ref[...] = jax.nn.sigmoid(rm[:, :3])


def kernel(origins, directions, appearance_embedding, centroids,
           W1, b1, W2, b2, W3, b3):
    n = origins.shape[0]
    o = origins[:, 0, :]
    d = directions[:, 0, :]
    a = appearance_embedding[:, 0, :]

    cmat = jnp.full((8, 128), 1e9, jnp.float32)
    cmat = cmat.at[:3, :E].set(centroids.T)

    w1cat = W1.transpose(1, 0, 2).reshape(W1.shape[1], WIDE)
    w1d = w1cat[:3]
    w1a = w1cat[3:]
    b1cat = b1.reshape(1, WIDE)

    w2s = W2.reshape(WIDE, HID)
    b2p = jnp.pad(b2, ((0, 128 - E), (0, 0)))
    w3c = jnp.pad(W3, ((0, 0), (0, 0), (0, OUT_G - 3))) \
        .transpose(1, 0, 2).reshape(HID, E * OUT_G)
    b3c = jnp.pad(b3, ((0, 0), (0, OUT_G - 3))).reshape(1, E * OUT_G)

    grid = (n // BLK,)
    out = pl.pallas_call(
        _moe_body,
        grid=grid,
        in_specs=[
            pl.BlockSpec((BLK, 3), lambda i: (i, 0)),
            pl.BlockSpec((BLK, 3), lambda i: (i, 0)),
            pl.BlockSpec((BLK, 32), lambda i: (i, 0)),
            pl.BlockSpec((8, 128), lambda i: (0, 0)),
            pl.BlockSpec((3, WIDE), lambda i: (0, 0)),
            pl.BlockSpec((32, WIDE), lambda i: (0, 0)),
            pl.BlockSpec((1, WIDE), lambda i: (0, 0)),
            pl.BlockSpec((WIDE, HID), lambda i: (0, 0)),
            pl.BlockSpec((128, HID), lambda i: (0, 0)),
            pl.BlockSpec((HID, 128), lambda i: (0, 0)),
            pl.BlockSpec((1, 128), lambda i: (0, 0)),
        ],
        out_specs=pl.BlockSpec((BLK, 3), lambda i: (i, 0)),
        out_shape=jax.ShapeDtypeStruct((n, 3), jnp.float32),
    )(o, d, a, cmat, w1d, w1a, b1cat, w2s, b2p, w3c, b3c)
    return out


# speedup vs baseline: 26.0246x; 1.0825x over previous
import jax
import jax.numpy as jnp
from jax.experimental import pallas as pl

E = 16
HID = 64
WIDE = E * HID
OUT_G = 8
BLK = 1024


def _moe_body(o_ref, d_ref, a_ref, cmat_ref, w1d_ref, w1a_ref, b1_ref,
              w2s_ref, b2p_ref, w3c_ref, b3c_ref, out_ref):
    o = o_ref[...]
    d = d_ref[...]
    a = a_ref[...]

    score = jnp.zeros((BLK, 128), jnp.float32)
    for k in range(3):
        dd = o[:, k:k + 1] - cmat_ref[k:k + 1, :]
        score = score + dd * dd
    cluster = jnp.argmin(score, axis=1).astype(jnp.int32)[:, None]

    lane128 = jax.lax.broadcasted_iota(jnp.int32, (BLK, 128), 1)
    onehot = (lane128 == cluster).astype(jnp.float32)
    eidx = jax.lax.broadcasted_iota(jnp.int32, (BLK, WIDE), 1) // HID
    mask = (eidx == cluster).astype(jnp.float32)

    h1 = jnp.maximum(
        jnp.dot(d, w1d_ref[...], preferred_element_type=jnp.float32)
        + jnp.dot(a, w1a_ref[...], preferred_element_type=jnp.float32)
        + b1_ref[...], 0.0)
    h2 = jnp.maximum(
        jnp.dot(h1 * mask, w2s_ref[...], preferred_element_type=jnp.float32)
        + jnp.dot(onehot, b2p_ref[...], preferred_element_type=jnp.float32),
        0.0)
    r = jnp.dot(h2, w3c_ref[...], preferred_element_type=jnp.float32) \
        + b3c_ref[...]
    mask8 = (lane128 // OUT_G == cluster).astype(jnp.float32)
    rm = r * mask8
    rm = rm[:, :64] + rm[:, 64:]
    rm = rm[:, :32] + rm[:, 32:]
    rm = rm[:, :16] + rm[:, 16:]
    rm = rm[:, :8] + rm[:, 8:]
    out_ref[...] = jax.nn.sigmoid(rm[:, :3])


def kernel(origins, directions, appearance_embedding, centroids,
           W1, b1, W2, b2, W3, b3):
    n = origins.shape[0]
    o = origins[:, 0, :]
    d = directions[:, 0, :]
    a = appearance_embedding[:, 0, :]

    cmat = jnp.full((8, 128), 1e9, jnp.float32)
    cmat = cmat.at[:3, :E].set(centroids.T)

    w1cat = W1.transpose(1, 0, 2).reshape(W1.shape[1], WIDE)
    w1d = w1cat[:3]
    w1a = w1cat[3:]
    b1cat = b1.reshape(1, WIDE)

    w2s = W2.reshape(WIDE, HID)
    b2p = jnp.pad(b2, ((0, 128 - E), (0, 0)))
    w3c = jnp.pad(W3, ((0, 0), (0, 0), (0, OUT_G - 3))) \
        .transpose(1, 0, 2).reshape(HID, E * OUT_G)
    b3c = jnp.pad(b3, ((0, 0), (0, OUT_G - 3))).reshape(1, E * OUT_G)

    grid = (n // BLK,)
    out = pl.pallas_call(
        _moe_body,
        grid=grid,
        in_specs=[
            pl.BlockSpec((BLK, 3), lambda i: (i, 0)),
            pl.BlockSpec((BLK, 3), lambda i: (i, 0)),
            pl.BlockSpec((BLK, 32), lambda i: (i, 0)),
            pl.BlockSpec((8, 128), lambda i: (0, 0)),
            pl.BlockSpec((3, WIDE), lambda i: (0, 0)),
            pl.BlockSpec((32, WIDE), lambda i: (0, 0)),
            pl.BlockSpec((1, WIDE), lambda i: (0, 0)),
            pl.BlockSpec((WIDE, HID), lambda i: (0, 0)),
            pl.BlockSpec((128, HID), lambda i: (0, 0)),
            pl.BlockSpec((HID, 128), lambda i: (0, 0)),
            pl.BlockSpec((1, 128), lambda i: (0, 0)),
        ],
        out_specs=pl.BlockSpec((BLK, 3), lambda i: (i, 0)),
        out_shape=jax.ShapeDtypeStruct((n, 3), jnp.float32),
    )(o, d, a, cmat, w1d, w1a, b1cat, w2s, b2p, w3c, b3c)
    return out
